# baseline (speedup 1.0000x reference)
import jax
import jax.numpy as jnp
from jax.experimental import pallas as pl

B = 16384
N = 1000
COLS = 1024


def _onehot_body(x_ref, g_ref, o_ref):
    t = x_ref[...] + g_ref[...]
    m = jnp.max(t, axis=0, keepdims=True)
    e = jnp.exp(t - m)
    s = jnp.sum(e, axis=0, keepdims=True)
    y = e / s
    m2 = 1.0 / s
    row = jax.lax.broadcasted_iota(jnp.int32, t.shape, 0)
    first = jnp.min(jnp.where(y == m2, row, N), axis=0, keepdims=True)
    val = (1.0 - m2) + m2
    val = jnp.where(val > 0.5, val, 0.0)
    o_ref[...] = jnp.where(row == first, val, 0.0)

    @pl.when(pl.program_id(0) == 0)
    def _():
        o_ref[1:2, 0:1] = jnp.ones((1, 1), jnp.float32)


@jax.jit
def kernel(x, gumbels):
    out_t = pl.pallas_call(
        _onehot_body,
        grid=(B // COLS,),
        in_specs=[
            pl.BlockSpec((N, COLS), lambda i: (0, i)),
            pl.BlockSpec((N, COLS), lambda i: (0, i)),
        ],
        out_specs=pl.BlockSpec((N, COLS), lambda i: (0, i)),
        out_shape=jax.ShapeDtypeStruct((N, B), jnp.float32),
    )(x.T, gumbels.T)
    return out_t.T


# speedup vs baseline: 1.0074x; 1.0074x over previous
import jax
import jax.numpy as jnp
from jax.experimental import pallas as pl

B = 16384
N = 1000
COLS = 1024


def _next_f32(c):
    b = jax.lax.bitcast_convert_type(c, jnp.int32)
    return jax.lax.bitcast_convert_type(b + 1, jnp.float32)


def _prev_f32(c):
    b = jax.lax.bitcast_convert_type(c, jnp.int32)
    return jax.lax.bitcast_convert_type(b - 1, jnp.float32)


def _onehot_body(x_ref, g_ref, o_ref):
    t = x_ref[...] + g_ref[...]
    m = jnp.max(t, axis=0, keepdims=True)
    e = jnp.exp(t - m)
    s = jnp.sum(e, axis=0, keepdims=True)
    m2 = 1.0 / s
    c = m2 * s
    for _ in range(3):
        c = jnp.where((c / s) < m2, _next_f32(c), c)
    for _ in range(3):
        cd = _prev_f32(c)
        c = jnp.where((cd / s) >= m2, cd, c)
    row = jax.lax.broadcasted_iota(jnp.int32, t.shape, 0)
    first = jnp.min(jnp.where(e >= c, row, N), axis=0, keepdims=True)
    val = (1.0 - m2) + m2
    val = jnp.where(val > 0.5, val, 0.0)
    o_ref[...] = jnp.where(row == first, val, 0.0)

    @pl.when(pl.program_id(0) == 0)
    def _():
        o_ref[1:2, 0:1] = jnp.ones((1, 1), jnp.float32)


@jax.jit
def kernel(x, gumbels):
    out_t = pl.pallas_call(
        _onehot_body,
        grid=(B // COLS,),
        in_specs=[
            pl.BlockSpec((N, COLS), lambda i: (0, i)),
            pl.BlockSpec((N, COLS), lambda i: (0, i)),
        ],
        out_specs=pl.BlockSpec((N, COLS), lambda i: (0, i)),
        out_shape=jax.ShapeDtypeStruct((N, B), jnp.float32),
    )(x.T, gumbels.T)
    return out_t.T
